# baseline (speedup 1.0000x reference)
import functools

import jax
import jax.numpy as jnp
from jax import lax
from jax.experimental import pallas as pl
from jax.experimental.pallas import tpu as pltpu
from jax.experimental.pallas import tpu_sc as plsc

_N = 10000
_E = 320000
_H = 128

_NC = 2
_NS = 16
_NW = _NC * _NS
_CH = 80

_BE = 5000


def _lrelu(x):
    return jnp.where(x > 0, x, 0.01 * x)


def _node_init_body(xk, emb, w1, b1, w2, b2, w3, b3, wa, wb, nf, a, b):
    m = (xk[...] == 0).astype(jnp.float32)
    feat = m * emb[0:1, :] + (1.0 - m) * emb[1:2, :]
    h = _lrelu(jnp.dot(feat, w1[...], preferred_element_type=jnp.float32) + b1[...])
    h = _lrelu(jnp.dot(h, w2[...], preferred_element_type=jnp.float32) + b2[...])
    nfv = jnp.dot(h, w3[...], preferred_element_type=jnp.float32) + b3[...]
    nf[...] = nfv
    a[...] = jnp.dot(nfv, wa[...], preferred_element_type=jnp.float32)
    b[...] = jnp.dot(nfv, wb[...], preferred_element_type=jnp.float32)


def _edge_mlp_body(write_ef, g1, g2, ef, nd, w1c, b1, w2, b2, w3, b3, *outs):
    h = (g1[...] + g2[...]
         + jnp.dot(ef[...], w1c[...], preferred_element_type=jnp.float32)
         + b1[...])
    h = _lrelu(h)
    h = _lrelu(jnp.dot(h, w2[...], preferred_element_type=jnp.float32) + b2[...])
    msg = jnp.dot(h, w3[...], preferred_element_type=jnp.float32) + b3[...]
    if write_ef:
        outs[0][...] = ef[...] + msg
        outs[1][...] = msg * nd[...]
    else:
        outs[0][...] = msg * nd[...]


def _edge0_body(g1, g2, ea, nd, i1, d1, i2, d2, i3, d3,
                w1c, b1, w2, b2, w3, b3, efo, msgd):
    ef = _lrelu(jnp.dot(ea[...], i1[...], preferred_element_type=jnp.float32) + d1[...])
    ef = _lrelu(jnp.dot(ef, i2[...], preferred_element_type=jnp.float32) + d2[...])
    ef = jnp.dot(ef, i3[...], preferred_element_type=jnp.float32) + d3[...]
    h = (g1[...] + g2[...]
         + jnp.dot(ef, w1c[...], preferred_element_type=jnp.float32)
         + b1[...])
    h = _lrelu(h)
    h = _lrelu(jnp.dot(h, w2[...], preferred_element_type=jnp.float32) + b2[...])
    msg = jnp.dot(h, w3[...], preferred_element_type=jnp.float32) + b3[...]
    efo[...] = ef + msg
    msgd[...] = msg * nd[...]


def _node_upd_body(nf, p, v1a, v1b, c1, v2, c2, v3, c3, wa, wb, nfo, a, b):
    aggr = p[0] + p[1]
    h = _lrelu(jnp.dot(nf[...], v1a[...], preferred_element_type=jnp.float32)
               + jnp.dot(aggr, v1b[...], preferred_element_type=jnp.float32) + c1[...])
    h = _lrelu(jnp.dot(h, v2[...], preferred_element_type=jnp.float32) + c2[...])
    nfv = nf[...] + jnp.dot(h, v3[...], preferred_element_type=jnp.float32) + c3[...]
    nfo[...] = nfv
    a[...] = jnp.dot(nfv, wa[...], preferred_element_type=jnp.float32)
    b[...] = jnp.dot(nfv, wb[...], preferred_element_type=jnp.float32)


def _node_final_body(nf, p, v1a, v1b, c1, v2, c2, v3, c3,
                     o1, d1, o2, d2, o3, d3, out):
    aggr = p[0] + p[1]
    h = _lrelu(jnp.dot(nf[...], v1a[...], preferred_element_type=jnp.float32)
               + jnp.dot(aggr, v1b[...], preferred_element_type=jnp.float32) + c1[...])
    h = _lrelu(jnp.dot(h, v2[...], preferred_element_type=jnp.float32) + c2[...])
    nfv = nf[...] + jnp.dot(h, v3[...], preferred_element_type=jnp.float32) + c3[...]
    h = _lrelu(jnp.dot(nfv, o1[...], preferred_element_type=jnp.float32) + d1[...])
    h = _lrelu(jnp.dot(h, o2[...], preferred_element_type=jnp.float32) + d2[...])
    out[...] = jnp.dot(h, o3[...], preferred_element_type=jnp.float32) + d3[...]


def _const_spec(shape):
    return pl.BlockSpec(shape, lambda i: (0,) * len(shape))


def _edge_mlp_call(write_ef, g1, g2, ef, nd, w1c, b1, w2, b2, w3, b3):
    ne = g1.shape[0]
    grid = (ne // _BE,)
    blk = pl.BlockSpec((_BE, _H), lambda i: (i, 0))
    gblk = pl.BlockSpec((_BE, _H), lambda i: (i, 0))
    n_out = 2 if write_ef else 1
    res = pl.pallas_call(
        functools.partial(_edge_mlp_body, write_ef),
        grid=grid,
        in_specs=[
            gblk, gblk, blk,
            pl.BlockSpec((_BE, 1), lambda i: (i, 0)),
            _const_spec(w1c.shape), _const_spec(b1.shape),
            _const_spec(w2.shape), _const_spec(b2.shape),
            _const_spec(w3.shape), _const_spec(b3.shape),
        ],
        out_specs=[blk] * n_out,
        out_shape=[jax.ShapeDtypeStruct((ne, _H), jnp.float32)] * n_out,
        compiler_params=pltpu.CompilerParams(
            dimension_semantics=("arbitrary",)),
    )(g1, g2, ef, nd, w1c, b1, w2, b2, w3, b3)
    return res if write_ef else res[0]


def _edge0_call(g1, g2, ea, nd, ei, lw):
    ne = g1.shape[0]
    grid = (ne // _BE,)
    blk = pl.BlockSpec((_BE, _H), lambda i: (i, 0))
    gblk = pl.BlockSpec((_BE, _H), lambda i: (i, 0))
    consts = [ei[0], ei[1], ei[2], ei[3], ei[4], ei[5],
              lw["w1c"], lw["b1"], lw["w2"], lw["b2"], lw["w3"], lw["b3"]]
    return pl.pallas_call(
        _edge0_body,
        grid=grid,
        in_specs=[
            gblk, gblk,
            pl.BlockSpec((_BE, ea.shape[1]), lambda i: (i, 0)),
            pl.BlockSpec((_BE, 1), lambda i: (i, 0)),
        ] + [_const_spec(c.shape) for c in consts],
        out_specs=[blk, blk],
        out_shape=[jax.ShapeDtypeStruct((ne, _H), jnp.float32)] * 2,
        compiler_params=pltpu.CompilerParams(
            dimension_semantics=("arbitrary",)),
    )(g1, g2, ea, nd, *consts)


_UN = 5
_NSTR = 1
_ES = _E // _NSTR


def _sc_gather(a, b, dst, src):
    mesh = plsc.VectorSubcoreMesh(core_axis_name="c", subcore_axis_name="s")
    ne = dst.shape[0]
    epw = ne // _NW
    nch = epw // _CH
    nphase = nch // _UN

    gdt = a.dtype
    gw = a.shape[1]

    @functools.partial(
        pl.kernel,
        out_type=[jax.ShapeDtypeStruct((ne, gw), gdt),
                  jax.ShapeDtypeStruct((ne, gw), gdt)],
        mesh=mesh,
        scratch_types=(
            [pltpu.VMEM((epw,), jnp.int32)] * 2
            + [pltpu.VMEM((_CH, gw), gdt)] * (2 * _UN)
            + [pltpu.SemaphoreType.DMA] * (4 * _UN)
        ),
    )
    def gather_k(a_hbm, b_hbm, dst_hbm, src_hbm, g1_hbm, g2_hbm, *scr):
        idxd, idxs = scr[0], scr[1]
        bufa = scr[2:2 + _UN]
        bufb = scr[2 + _UN:2 + 2 * _UN]
        gsa = scr[2 + 2 * _UN:2 + 3 * _UN]
        gsb = scr[2 + 3 * _UN:2 + 4 * _UN]
        wsa = scr[2 + 4 * _UN:2 + 5 * _UN]
        wsb = scr[2 + 5 * _UN:2 + 6 * _UN]
        wid = lax.axis_index("s") * _NC + lax.axis_index("c")
        base = wid * epw
        pltpu.sync_copy(dst_hbm.at[pl.ds(base, epw)], idxd)
        pltpu.sync_copy(src_hbm.at[pl.ds(base, epw)], idxs)

        def fire(s, ch):
            pltpu.async_copy(a_hbm.at[idxd.at[pl.ds(ch * _CH, _CH)]],
                             bufa[s], gsa[s])
            pltpu.async_copy(b_hbm.at[idxs.at[pl.ds(ch * _CH, _CH)]],
                             bufb[s], gsb[s])

        def drain_and_write(s, ch):
            off = base + ch * _CH
            pltpu.make_async_copy(a_hbm.at[idxd.at[pl.ds(0, _CH)]],
                                  bufa[s], gsa[s]).wait()
            pltpu.make_async_copy(b_hbm.at[idxs.at[pl.ds(0, _CH)]],
                                  bufb[s], gsb[s]).wait()
            pltpu.async_copy(bufa[s], g1_hbm.at[pl.ds(off, _CH)], wsa[s])
            pltpu.async_copy(bufb[s], g2_hbm.at[pl.ds(off, _CH)], wsb[s])

        def drain_write(s):
            pltpu.make_async_copy(bufa[s], g1_hbm.at[pl.ds(base, _CH)],
                                  wsa[s]).wait()
            pltpu.make_async_copy(bufb[s], g2_hbm.at[pl.ds(base, _CH)],
                                  wsb[s]).wait()

        for s in range(_UN):
            fire(s, s)

        def body(j, carry):
            for s in range(_UN):
                drain_and_write(s, j * _UN + s)
            for s in range(_UN):
                drain_write(s)
                fire(s, (j + 1) * _UN + s)
            return carry

        lax.fori_loop(0, nphase - 1, body, 0)
        for s in range(_UN):
            drain_and_write(s, (nphase - 1) * _UN + s)
        for s in range(_UN):
            drain_write(s)

    return gather_k(a, b, dst, src)


def _sc_scatter(msgds, dsts, zeros):
    mesh = plsc.VectorSubcoreMesh(core_axis_name="c", subcore_axis_name="s")
    nstr = len(msgds)
    epc = _ES // _NC
    ept = epc // _NS
    nchunks = _N // _CH
    kmax = (nchunks + _NS - 1) // _NS

    ncht = ept // _CH
    UNS = 4
    nfull = ncht // UNS - 1

    @functools.partial(
        pl.kernel,
        out_type=jax.ShapeDtypeStruct((_NC, _N, _H), jnp.float32),
        mesh=mesh,
        scratch_types=(
            [pltpu.VMEM((_CH,), jnp.int32)] * UNS
            + [pltpu.VMEM((_CH, _H), jnp.float32)] * UNS
            + [pltpu.VMEM_SHARED((_N, _H), jnp.float32)]
            + [pltpu.SemaphoreType.DMA] * (3 * UNS + 3)
        ),
    )
    def scatter_k(*refs):
        msgd_hbm = refs[0:nstr]
        dst_hbm = refs[nstr:2 * nstr]
        zeros_hbm = refs[2 * nstr]
        part_hbm = refs[2 * nstr + 1]
        scr = refs[2 * nstr + 2:]
        idx = scr[0:UNS]
        buf = scr[UNS:2 * UNS]
        accum = scr[2 * UNS]
        isem = scr[2 * UNS + 1:3 * UNS + 1]
        dsem = scr[3 * UNS + 1:4 * UNS + 1]
        ssem = scr[4 * UNS + 1:5 * UNS + 1]
        zsem = scr[5 * UNS + 1]
        rsem = (scr[5 * UNS + 2], scr[5 * UNS + 3])
        cid = lax.axis_index("c")
        sid = lax.axis_index("s")

        pltpu.sync_copy(zeros_hbm, buf[0])
        for k in range(kmax):
            ch = sid + k * _NS

            @pl.when(ch < nchunks)
            def _():
                pltpu.async_copy(buf[0], accum.at[pl.ds(ch * _CH, _CH)], zsem)

        for k in range(kmax):
            ch = sid + k * _NS

            @pl.when(ch < nchunks)
            def _():
                pltpu.make_async_copy(buf[0], accum.at[pl.ds(0, _CH)],
                                      zsem).wait()

        plsc.subcore_barrier()

        ebase = cid * epc + sid * ept

        def run_stripe(mref, dref):
            def fire_load(s, ch):
                off = ebase + ch * _CH
                pltpu.async_copy(dref.at[pl.ds(off, _CH)], idx[s], isem[s])
                pltpu.async_copy(mref.at[pl.ds(off, _CH)], buf[s], dsem[s])

            def drain_and_scatter(s):
                pltpu.make_async_copy(dref.at[pl.ds(ebase, _CH)],
                                      idx[s], isem[s]).wait()
                pltpu.make_async_copy(mref.at[pl.ds(ebase, _CH)],
                                      buf[s], dsem[s]).wait()
                pltpu.async_copy(buf[s], accum.at[idx[s]], ssem[s], add=True)

            def drain_scatter(s):
                pltpu.make_async_copy(buf[s], accum.at[idx[s]], ssem[s]).wait()

            for s in range(UNS):
                fire_load(s, s)

            def body(j, carry):
                for s in range(UNS):
                    drain_and_scatter(s)
                for s in range(UNS):
                    drain_scatter(s)
                    fire_load(s, (j + 1) * UNS + s)
                return carry

            lax.fori_loop(0, nfull, body, 0)
            for s in range(UNS):
                drain_and_scatter(s)
            drain_scatter(0)
            fire_load(0, ncht - 1)
            drain_and_scatter(0)
            for s in range(UNS):
                drain_scatter(s)

        for st in range(nstr):
            run_stripe(msgd_hbm[st], dst_hbm[st])
        plsc.subcore_barrier()

        for k in range(kmax):
            ch = sid + k * _NS
            p = k % 2

            @pl.when(ch < nchunks)
            def _():
                r = ch * _CH
                if k >= 2:
                    pltpu.make_async_copy(
                        buf[p], part_hbm.at[cid].at[pl.ds(0, _CH)],
                        rsem[p]).wait()
                pltpu.sync_copy(accum.at[pl.ds(r, _CH)], buf[p])
                pltpu.async_copy(buf[p], part_hbm.at[cid].at[pl.ds(r, _CH)],
                                 rsem[p])

        for k in range(kmax):
            ch = sid + k * _NS

            @pl.when(jnp.logical_and(ch < nchunks,
                                     ch + 2 * _NS >= nchunks))
            def _():
                pltpu.make_async_copy(
                    buf[k % 2],
                    part_hbm.at[cid].at[pl.ds(0, _CH)], rsem[k % 2]).wait()

    return scatter_k(*msgds, *dsts, zeros)


def _rb(b):
    return b.reshape(1, -1)


def kernel(x, edge_index, edge_attr, node_dist, params):
    src = edge_index[0].astype(jnp.int32)
    dst = edge_index[1].astype(jnp.int32)
    x2 = x.astype(jnp.int32).reshape(_N, 1)
    emb = params["embed"]

    (niw1, nib1), (niw2, nib2), (niw3, nib3) = params["node_in"]
    (eiw1, eib1), (eiw2, eib2), (eiw3, eib3) = params["edge_in"]
    (now1, nod1), (now2, nod2), (now3, nod3) = params["node_out"]

    layer_w = []
    for lp in params["layers"]:
        (w1, b1), (w2, b2), (w3, b3) = lp["lin_edge"]
        (v1, c1), (v2, c2), (v3, c3) = lp["lin_node"]
        layer_w.append(dict(
            wa=w1[0:_H], wb=w1[_H:2 * _H], w1c=w1[2 * _H:3 * _H],
            b1=_rb(b1), w2=w2, b2=_rb(b2), w3=w3, b3=_rb(b3),
            v1a=v1[0:_H], v1b=v1[_H:2 * _H], c1=_rb(c1),
            v2=v2, c2=_rb(c2), v3=v3, c3=_rb(c3),
        ))

    _node_out3 = [jax.ShapeDtypeStruct((_N, _H), jnp.float32)] * 3
    nf, ga, gb = pl.pallas_call(
        _node_init_body,
        out_shape=_node_out3,
    )(x2, emb, niw1, _rb(nib1), niw2, _rb(nib2), niw3, _rb(nib3),
      layer_w[0]["wa"], layer_w[0]["wb"])

    zeros = jnp.zeros((_CH, _H), jnp.float32)
    nd = node_dist.astype(jnp.float32)

    dst_s = [lax.slice(dst, (i * _ES,), ((i + 1) * _ES,))
             for i in range(_NSTR)]
    src_s = [lax.slice(src, (i * _ES,), ((i + 1) * _ES,))
             for i in range(_NSTR)]
    ea_s = [lax.slice(edge_attr, (i * _ES, 0), ((i + 1) * _ES, edge_attr.shape[1]))
            for i in range(_NSTR)]
    nd_s = [lax.slice(nd, (i * _ES, 0), ((i + 1) * _ES, 1))
            for i in range(_NSTR)]

    ei = (eiw1, _rb(eib1), eiw2, _rb(eib2), eiw3, _rb(eib3))
    ef_s = [None] * _NSTR

    for l, lw in enumerate(layer_w):
        last = l == len(layer_w) - 1
        g_s = [_sc_gather(ga, gb, dst_s[i], src_s[i]) for i in range(_NSTR)]
        msgd_s = []
        for i in range(_NSTR):
            g1, g2 = g_s[i]
            if l == 0:
                ef_s[i], msgd = _edge0_call(g1, g2, ea_s[i], nd_s[i], ei, lw)
            elif not last:
                ef_s[i], msgd = _edge_mlp_call(
                    True, g1, g2, ef_s[i], nd_s[i],
                    lw["w1c"], lw["b1"], lw["w2"], lw["b2"],
                    lw["w3"], lw["b3"])
            else:
                msgd = _edge_mlp_call(
                    False, g1, g2, ef_s[i], nd_s[i],
                    lw["w1c"], lw["b1"], lw["w2"], lw["b2"],
                    lw["w3"], lw["b3"])
            msgd_s.append(msgd)
        p = _sc_scatter(msgd_s, dst_s, zeros)
        if not last:
            nxt = layer_w[l + 1]
            nf, ga, gb = pl.pallas_call(
                _node_upd_body,
                out_shape=_node_out3,
            )(nf, p, lw["v1a"], lw["v1b"], lw["c1"], lw["v2"], lw["c2"],
              lw["v3"], lw["c3"], nxt["wa"], nxt["wb"])
        else:
            out = pl.pallas_call(
                _node_final_body,
                out_shape=jax.ShapeDtypeStruct((_N, now3.shape[1]), jnp.float32),
            )(nf, p, lw["v1a"], lw["v1b"], lw["c1"], lw["v2"], lw["c2"],
              lw["v3"], lw["c3"], now1, _rb(nod1), now2, _rb(nod2),
              now3, _rb(nod3))
    return out


# speedup vs baseline: 1.0123x; 1.0123x over previous
import functools

import jax
import jax.numpy as jnp
from jax import lax
from jax.experimental import pallas as pl
from jax.experimental.pallas import tpu as pltpu
from jax.experimental.pallas import tpu_sc as plsc

_N = 10000
_E = 320000
_H = 128

_NC = 2
_NS = 16
_NW = _NC * _NS
_CH = 80

_BE = 4000
_BN = 2000


def _lrelu(x):
    return jnp.where(x > 0, x, 0.01 * x)


def _node_init_body(xk, emb, w1, b1, w2, b2, w3, b3, wa, wb, nf, a, b):
    m = (xk[...] == 0).astype(jnp.float32)
    feat = m * emb[0:1, :] + (1.0 - m) * emb[1:2, :]
    h = _lrelu(jnp.dot(feat, w1[...], preferred_element_type=jnp.float32) + b1[...])
    h = _lrelu(jnp.dot(h, w2[...], preferred_element_type=jnp.float32) + b2[...])
    nfv = jnp.dot(h, w3[...], preferred_element_type=jnp.float32) + b3[...]
    nf[...] = nfv
    a[...] = jnp.dot(nfv, wa[...], preferred_element_type=jnp.float32)
    b[...] = jnp.dot(nfv, wb[...], preferred_element_type=jnp.float32)


def _edge_mlp_body(write_ef, g1, g2, ef, nd, w1c, b1, w2, b2, w3, b3, *outs):
    h = (g1[...] + g2[...]
         + jnp.dot(ef[...], w1c[...], preferred_element_type=jnp.float32)
         + b1[...])
    h = _lrelu(h)
    h = _lrelu(jnp.dot(h, w2[...], preferred_element_type=jnp.float32) + b2[...])
    msg = jnp.dot(h, w3[...], preferred_element_type=jnp.float32) + b3[...]
    if write_ef:
        outs[0][...] = ef[...] + msg
        outs[1][...] = msg * nd[...]
    else:
        outs[0][...] = msg * nd[...]


def _edge0_body(g1, g2, ea, nd, i1, d1, i2, d2, i3, d3,
                w1c, b1, w2, b2, w3, b3, efo, msgd):
    ef = _lrelu(jnp.dot(ea[...], i1[...], preferred_element_type=jnp.float32) + d1[...])
    ef = _lrelu(jnp.dot(ef, i2[...], preferred_element_type=jnp.float32) + d2[...])
    ef = jnp.dot(ef, i3[...], preferred_element_type=jnp.float32) + d3[...]
    h = (g1[...] + g2[...]
         + jnp.dot(ef, w1c[...], preferred_element_type=jnp.float32)
         + b1[...])
    h = _lrelu(h)
    h = _lrelu(jnp.dot(h, w2[...], preferred_element_type=jnp.float32) + b2[...])
    msg = jnp.dot(h, w3[...], preferred_element_type=jnp.float32) + b3[...]
    efo[...] = ef + msg
    msgd[...] = msg * nd[...]


def _node_upd_body(nf, p, v1a, v1b, c1, v2, c2, v3, c3, wa, wb, nfo, a, b):
    aggr = p[0] + p[1]
    h = _lrelu(jnp.dot(nf[...], v1a[...], preferred_element_type=jnp.float32)
               + jnp.dot(aggr, v1b[...], preferred_element_type=jnp.float32) + c1[...])
    h = _lrelu(jnp.dot(h, v2[...], preferred_element_type=jnp.float32) + c2[...])
    nfv = nf[...] + jnp.dot(h, v3[...], preferred_element_type=jnp.float32) + c3[...]
    nfo[...] = nfv
    a[...] = jnp.dot(nfv, wa[...], preferred_element_type=jnp.float32)
    b[...] = jnp.dot(nfv, wb[...], preferred_element_type=jnp.float32)


def _node_final_body(nf, p, v1a, v1b, c1, v2, c2, v3, c3,
                     o1, d1, o2, d2, o3, d3, out):
    aggr = p[0] + p[1]
    h = _lrelu(jnp.dot(nf[...], v1a[...], preferred_element_type=jnp.float32)
               + jnp.dot(aggr, v1b[...], preferred_element_type=jnp.float32) + c1[...])
    h = _lrelu(jnp.dot(h, v2[...], preferred_element_type=jnp.float32) + c2[...])
    nfv = nf[...] + jnp.dot(h, v3[...], preferred_element_type=jnp.float32) + c3[...]
    h = _lrelu(jnp.dot(nfv, o1[...], preferred_element_type=jnp.float32) + d1[...])
    h = _lrelu(jnp.dot(h, o2[...], preferred_element_type=jnp.float32) + d2[...])
    out[...] = jnp.dot(h, o3[...], preferred_element_type=jnp.float32) + d3[...]


def _const_spec(shape):
    return pl.BlockSpec(shape, lambda i: (0,) * len(shape))


def _edge_mlp_call(write_ef, g1, g2, ef, nd, w1c, b1, w2, b2, w3, b3):
    ne = g1.shape[0]
    grid = (ne // _BE,)
    blk = pl.BlockSpec((_BE, _H), lambda i: (i, 0))
    gblk = pl.BlockSpec((_BE, _H), lambda i: (i, 0))
    n_out = 2 if write_ef else 1
    res = pl.pallas_call(
        functools.partial(_edge_mlp_body, write_ef),
        grid=grid,
        in_specs=[
            gblk, gblk, blk,
            pl.BlockSpec((_BE, 1), lambda i: (i, 0)),
            _const_spec(w1c.shape), _const_spec(b1.shape),
            _const_spec(w2.shape), _const_spec(b2.shape),
            _const_spec(w3.shape), _const_spec(b3.shape),
        ],
        out_specs=[blk] * n_out,
        out_shape=[jax.ShapeDtypeStruct((ne, _H), jnp.float32)] * n_out,
        compiler_params=pltpu.CompilerParams(
            dimension_semantics=("arbitrary",)),
    )(g1, g2, ef, nd, w1c, b1, w2, b2, w3, b3)
    return res if write_ef else res[0]


def _edge0_call(g1, g2, ea, nd, ei, lw):
    ne = g1.shape[0]
    grid = (ne // _BE,)
    blk = pl.BlockSpec((_BE, _H), lambda i: (i, 0))
    gblk = pl.BlockSpec((_BE, _H), lambda i: (i, 0))
    consts = [ei[0], ei[1], ei[2], ei[3], ei[4], ei[5],
              lw["w1c"], lw["b1"], lw["w2"], lw["b2"], lw["w3"], lw["b3"]]
    return pl.pallas_call(
        _edge0_body,
        grid=grid,
        in_specs=[
            gblk, gblk,
            pl.BlockSpec((_BE, ea.shape[1]), lambda i: (i, 0)),
            pl.BlockSpec((_BE, 1), lambda i: (i, 0)),
        ] + [_const_spec(c.shape) for c in consts],
        out_specs=[blk, blk],
        out_shape=[jax.ShapeDtypeStruct((ne, _H), jnp.float32)] * 2,
        compiler_params=pltpu.CompilerParams(
            dimension_semantics=("arbitrary",)),
    )(g1, g2, ea, nd, *consts)


_UN = 5
_NSTR = 1
_ES = _E // _NSTR


def _sc_gather(a, b, dst, src):
    mesh = plsc.VectorSubcoreMesh(core_axis_name="c", subcore_axis_name="s")
    ne = dst.shape[0]
    epw = ne // _NW
    nch = epw // _CH
    nphase = nch // _UN

    gdt = a.dtype
    gw = a.shape[1]

    @functools.partial(
        pl.kernel,
        out_type=[jax.ShapeDtypeStruct((ne, gw), gdt),
                  jax.ShapeDtypeStruct((ne, gw), gdt)],
        mesh=mesh,
        scratch_types=(
            [pltpu.VMEM((epw,), jnp.int32)] * 2
            + [pltpu.VMEM((_CH, gw), gdt)] * (2 * _UN)
            + [pltpu.SemaphoreType.DMA] * (4 * _UN)
        ),
    )
    def gather_k(a_hbm, b_hbm, dst_hbm, src_hbm, g1_hbm, g2_hbm, *scr):
        idxd, idxs = scr[0], scr[1]
        bufa = scr[2:2 + _UN]
        bufb = scr[2 + _UN:2 + 2 * _UN]
        gsa = scr[2 + 2 * _UN:2 + 3 * _UN]
        gsb = scr[2 + 3 * _UN:2 + 4 * _UN]
        wsa = scr[2 + 4 * _UN:2 + 5 * _UN]
        wsb = scr[2 + 5 * _UN:2 + 6 * _UN]
        wid = lax.axis_index("s") * _NC + lax.axis_index("c")
        base = wid * epw
        pltpu.sync_copy(dst_hbm.at[pl.ds(base, epw)], idxd)
        pltpu.sync_copy(src_hbm.at[pl.ds(base, epw)], idxs)

        def fire(s, ch):
            pltpu.async_copy(a_hbm.at[idxd.at[pl.ds(ch * _CH, _CH)]],
                             bufa[s], gsa[s])
            pltpu.async_copy(b_hbm.at[idxs.at[pl.ds(ch * _CH, _CH)]],
                             bufb[s], gsb[s])

        def drain_and_write(s, ch):
            off = base + ch * _CH
            pltpu.make_async_copy(a_hbm.at[idxd.at[pl.ds(0, _CH)]],
                                  bufa[s], gsa[s]).wait()
            pltpu.make_async_copy(b_hbm.at[idxs.at[pl.ds(0, _CH)]],
                                  bufb[s], gsb[s]).wait()
            pltpu.async_copy(bufa[s], g1_hbm.at[pl.ds(off, _CH)], wsa[s])
            pltpu.async_copy(bufb[s], g2_hbm.at[pl.ds(off, _CH)], wsb[s])

        def drain_write(s):
            pltpu.make_async_copy(bufa[s], g1_hbm.at[pl.ds(base, _CH)],
                                  wsa[s]).wait()
            pltpu.make_async_copy(bufb[s], g2_hbm.at[pl.ds(base, _CH)],
                                  wsb[s]).wait()

        for s in range(_UN):
            fire(s, s)

        def body(j, carry):
            for s in range(_UN):
                drain_and_write(s, j * _UN + s)
            for s in range(_UN):
                drain_write(s)
                fire(s, (j + 1) * _UN + s)
            return carry

        lax.fori_loop(0, nphase - 1, body, 0)
        for s in range(_UN):
            drain_and_write(s, (nphase - 1) * _UN + s)
        for s in range(_UN):
            drain_write(s)

    return gather_k(a, b, dst, src)


def _sc_scatter(msgds, dsts, zeros):
    mesh = plsc.VectorSubcoreMesh(core_axis_name="c", subcore_axis_name="s")
    nstr = len(msgds)
    epc = _ES // _NC
    ept = epc // _NS
    nchunks = _N // _CH
    kmax = (nchunks + _NS - 1) // _NS

    ncht = ept // _CH
    UNS = 4
    nfull = ncht // UNS - 1

    @functools.partial(
        pl.kernel,
        out_type=jax.ShapeDtypeStruct((_NC, _N, _H), jnp.float32),
        mesh=mesh,
        scratch_types=(
            [pltpu.VMEM((_CH,), jnp.int32)] * UNS
            + [pltpu.VMEM((_CH, _H), jnp.float32)] * UNS
            + [pltpu.VMEM_SHARED((_N, _H), jnp.float32)]
            + [pltpu.SemaphoreType.DMA] * (3 * UNS + 3)
        ),
    )
    def scatter_k(*refs):
        msgd_hbm = refs[0:nstr]
        dst_hbm = refs[nstr:2 * nstr]
        zeros_hbm = refs[2 * nstr]
        part_hbm = refs[2 * nstr + 1]
        scr = refs[2 * nstr + 2:]
        idx = scr[0:UNS]
        buf = scr[UNS:2 * UNS]
        accum = scr[2 * UNS]
        isem = scr[2 * UNS + 1:3 * UNS + 1]
        dsem = scr[3 * UNS + 1:4 * UNS + 1]
        ssem = scr[4 * UNS + 1:5 * UNS + 1]
        zsem = scr[5 * UNS + 1]
        rsem = (scr[5 * UNS + 2], scr[5 * UNS + 3])
        cid = lax.axis_index("c")
        sid = lax.axis_index("s")

        pltpu.sync_copy(zeros_hbm, buf[0])
        for k in range(kmax):
            ch = sid + k * _NS

            @pl.when(ch < nchunks)
            def _():
                pltpu.async_copy(buf[0], accum.at[pl.ds(ch * _CH, _CH)], zsem)

        for k in range(kmax):
            ch = sid + k * _NS

            @pl.when(ch < nchunks)
            def _():
                pltpu.make_async_copy(buf[0], accum.at[pl.ds(0, _CH)],
                                      zsem).wait()

        plsc.subcore_barrier()

        ebase = cid * epc + sid * ept

        def run_stripe(mref, dref):
            def fire_load(s, ch):
                off = ebase + ch * _CH
                pltpu.async_copy(dref.at[pl.ds(off, _CH)], idx[s], isem[s])
                pltpu.async_copy(mref.at[pl.ds(off, _CH)], buf[s], dsem[s])

            def drain_and_scatter(s):
                pltpu.make_async_copy(dref.at[pl.ds(ebase, _CH)],
                                      idx[s], isem[s]).wait()
                pltpu.make_async_copy(mref.at[pl.ds(ebase, _CH)],
                                      buf[s], dsem[s]).wait()
                pltpu.async_copy(buf[s], accum.at[idx[s]], ssem[s], add=True)

            def drain_scatter(s):
                pltpu.make_async_copy(buf[s], accum.at[idx[s]], ssem[s]).wait()

            for s in range(UNS):
                fire_load(s, s)

            def body(j, carry):
                for s in range(UNS):
                    drain_and_scatter(s)
                for s in range(UNS):
                    drain_scatter(s)
                    fire_load(s, (j + 1) * UNS + s)
                return carry

            lax.fori_loop(0, nfull, body, 0)
            for s in range(UNS):
                drain_and_scatter(s)
            drain_scatter(0)
            fire_load(0, ncht - 1)
            drain_and_scatter(0)
            for s in range(UNS):
                drain_scatter(s)

        for st in range(nstr):
            run_stripe(msgd_hbm[st], dst_hbm[st])
        plsc.subcore_barrier()

        for k in range(kmax):
            ch = sid + k * _NS
            p = k % 2

            @pl.when(ch < nchunks)
            def _():
                r = ch * _CH
                if k >= 2:
                    pltpu.make_async_copy(
                        buf[p], part_hbm.at[cid].at[pl.ds(0, _CH)],
                        rsem[p]).wait()
                pltpu.sync_copy(accum.at[pl.ds(r, _CH)], buf[p])
                pltpu.async_copy(buf[p], part_hbm.at[cid].at[pl.ds(r, _CH)],
                                 rsem[p])

        for k in range(kmax):
            ch = sid + k * _NS

            @pl.when(jnp.logical_and(ch < nchunks,
                                     ch + 2 * _NS >= nchunks))
            def _():
                pltpu.make_async_copy(
                    buf[k % 2],
                    part_hbm.at[cid].at[pl.ds(0, _CH)], rsem[k % 2]).wait()

    return scatter_k(*msgds, *dsts, zeros)


def _rb(b):
    return b.reshape(1, -1)


def kernel(x, edge_index, edge_attr, node_dist, params):
    src = edge_index[0].astype(jnp.int32)
    dst = edge_index[1].astype(jnp.int32)
    x2 = x.astype(jnp.int32).reshape(_N, 1)
    emb = params["embed"]

    (niw1, nib1), (niw2, nib2), (niw3, nib3) = params["node_in"]
    (eiw1, eib1), (eiw2, eib2), (eiw3, eib3) = params["edge_in"]
    (now1, nod1), (now2, nod2), (now3, nod3) = params["node_out"]

    layer_w = []
    for lp in params["layers"]:
        (w1, b1), (w2, b2), (w3, b3) = lp["lin_edge"]
        (v1, c1), (v2, c2), (v3, c3) = lp["lin_node"]
        layer_w.append(dict(
            wa=w1[0:_H], wb=w1[_H:2 * _H], w1c=w1[2 * _H:3 * _H],
            b1=_rb(b1), w2=w2, b2=_rb(b2), w3=w3, b3=_rb(b3),
            v1a=v1[0:_H], v1b=v1[_H:2 * _H], c1=_rb(c1),
            v2=v2, c2=_rb(c2), v3=v3, c3=_rb(c3),
        ))

    _node_out3 = [jax.ShapeDtypeStruct((_N, _H), jnp.float32)] * 3
    nblk = pl.BlockSpec((_BN, _H), lambda i: (i, 0))
    ngrid = (_N // _BN,)
    ncp = pltpu.CompilerParams(dimension_semantics=("arbitrary",))

    ni_consts = [emb, niw1, _rb(nib1), niw2, _rb(nib2), niw3, _rb(nib3),
                 layer_w[0]["wa"], layer_w[0]["wb"]]
    nf, ga, gb = pl.pallas_call(
        _node_init_body,
        grid=ngrid,
        in_specs=[pl.BlockSpec((_BN, 1), lambda i: (i, 0))]
                 + [_const_spec(c.shape) for c in ni_consts],
        out_specs=[nblk] * 3,
        out_shape=_node_out3,
        compiler_params=ncp,
    )(x2, *ni_consts)

    zeros = jnp.zeros((_CH, _H), jnp.float32)
    nd = node_dist.astype(jnp.float32)

    dst_s = [lax.slice(dst, (i * _ES,), ((i + 1) * _ES,))
             for i in range(_NSTR)]
    src_s = [lax.slice(src, (i * _ES,), ((i + 1) * _ES,))
             for i in range(_NSTR)]
    ea_s = [lax.slice(edge_attr, (i * _ES, 0), ((i + 1) * _ES, edge_attr.shape[1]))
            for i in range(_NSTR)]
    nd_s = [lax.slice(nd, (i * _ES, 0), ((i + 1) * _ES, 1))
            for i in range(_NSTR)]

    ei = (eiw1, _rb(eib1), eiw2, _rb(eib2), eiw3, _rb(eib3))
    ef_s = [None] * _NSTR

    for l, lw in enumerate(layer_w):
        last = l == len(layer_w) - 1
        g_s = [_sc_gather(ga, gb, dst_s[i], src_s[i]) for i in range(_NSTR)]
        msgd_s = []
        for i in range(_NSTR):
            g1, g2 = g_s[i]
            if l == 0:
                ef_s[i], msgd = _edge0_call(g1, g2, ea_s[i], nd_s[i], ei, lw)
            elif not last:
                ef_s[i], msgd = _edge_mlp_call(
                    True, g1, g2, ef_s[i], nd_s[i],
                    lw["w1c"], lw["b1"], lw["w2"], lw["b2"],
                    lw["w3"], lw["b3"])
            else:
                msgd = _edge_mlp_call(
                    False, g1, g2, ef_s[i], nd_s[i],
                    lw["w1c"], lw["b1"], lw["w2"], lw["b2"],
                    lw["w3"], lw["b3"])
            msgd_s.append(msgd)
        p = _sc_scatter(msgd_s, dst_s, zeros)
        pblk = pl.BlockSpec((_NC, _BN, _H), lambda i: (0, i, 0))
        if not last:
            nxt = layer_w[l + 1]
            nu_consts = [lw["v1a"], lw["v1b"], lw["c1"], lw["v2"], lw["c2"],
                         lw["v3"], lw["c3"], nxt["wa"], nxt["wb"]]
            nf, ga, gb = pl.pallas_call(
                _node_upd_body,
                grid=ngrid,
                in_specs=[nblk, pblk]
                         + [_const_spec(c.shape) for c in nu_consts],
                out_specs=[nblk] * 3,
                out_shape=_node_out3,
                compiler_params=ncp,
            )(nf, p, *nu_consts)
        else:
            nfin_consts = [lw["v1a"], lw["v1b"], lw["c1"], lw["v2"], lw["c2"],
                           lw["v3"], lw["c3"], now1, _rb(nod1), now2,
                           _rb(nod2), now3, _rb(nod3)]
            out = pl.pallas_call(
                _node_final_body,
                grid=ngrid,
                in_specs=[nblk, pblk]
                         + [_const_spec(c.shape) for c in nfin_consts],
                out_specs=pl.BlockSpec((_BN, now3.shape[1]), lambda i: (i, 0)),
                out_shape=jax.ShapeDtypeStruct((_N, now3.shape[1]), jnp.float32),
                compiler_params=ncp,
            )(nf, p, *nfin_consts)
    return out


# speedup vs baseline: 1.1253x; 1.1116x over previous
import functools

import jax
import jax.numpy as jnp
from jax import lax
from jax.experimental import pallas as pl
from jax.experimental.pallas import tpu as pltpu
from jax.experimental.pallas import tpu_sc as plsc

_N = 10000
_E = 320000
_H = 128

_NC = 2
_NS = 16
_NW = _NC * _NS
_CH = 80

_BE = 4000
_BN = 2000


def _lrelu(x):
    return jnp.where(x > 0, x, 0.01 * x)


def _node_init_body(xk, emb, w1, b1, w2, b2, w3, b3, wa, wb, nf, a, b):
    m = (xk[...] == 0).astype(jnp.float32)
    feat = m * emb[0:1, :] + (1.0 - m) * emb[1:2, :]
    h = _lrelu(jnp.dot(feat, w1[...], preferred_element_type=jnp.float32) + b1[...])
    h = _lrelu(jnp.dot(h, w2[...], preferred_element_type=jnp.float32) + b2[...])
    nfv = jnp.dot(h, w3[...], preferred_element_type=jnp.float32) + b3[...]
    nf[...] = nfv
    a[...] = jnp.dot(nfv, wa[...], preferred_element_type=jnp.float32)
    b[...] = jnp.dot(nfv, wb[...], preferred_element_type=jnp.float32)


def _ndcol(nd):
    return jnp.transpose(nd[...].reshape(1, -1), (1, 0))


def _edge_mlp_body(write_ef, g1, g2, ef, nd, w1c, b1, w2, b2, w3, b3, *outs):
    h = (g1[...] + g2[...]
         + jnp.dot(ef[...], w1c[...], preferred_element_type=jnp.float32)
         + b1[...])
    h = _lrelu(h)
    h = _lrelu(jnp.dot(h, w2[...], preferred_element_type=jnp.float32) + b2[...])
    msg = jnp.dot(h, w3[...], preferred_element_type=jnp.float32) + b3[...]
    if write_ef:
        outs[0][...] = ef[...] + msg
        outs[1][...] = msg * _ndcol(nd)
    else:
        outs[0][...] = msg * _ndcol(nd)


def _edge0_body(g1, g2, ea, nd, i1, d1, i2, d2, i3, d3,
                w1c, b1, w2, b2, w3, b3, efo, msgd):
    ef = _lrelu(jnp.dot(ea[...], i1[...], preferred_element_type=jnp.float32) + d1[...])
    ef = _lrelu(jnp.dot(ef, i2[...], preferred_element_type=jnp.float32) + d2[...])
    ef = jnp.dot(ef, i3[...], preferred_element_type=jnp.float32) + d3[...]
    h = (g1[...] + g2[...]
         + jnp.dot(ef, w1c[...], preferred_element_type=jnp.float32)
         + b1[...])
    h = _lrelu(h)
    h = _lrelu(jnp.dot(h, w2[...], preferred_element_type=jnp.float32) + b2[...])
    msg = jnp.dot(h, w3[...], preferred_element_type=jnp.float32) + b3[...]
    efo[...] = ef + msg
    msgd[...] = msg * _ndcol(nd)


def _node_upd_body(nf, p, v1a, v1b, c1, v2, c2, v3, c3, wa, wb, nfo, a, b):
    aggr = p[0] + p[1]
    h = _lrelu(jnp.dot(nf[...], v1a[...], preferred_element_type=jnp.float32)
               + jnp.dot(aggr, v1b[...], preferred_element_type=jnp.float32) + c1[...])
    h = _lrelu(jnp.dot(h, v2[...], preferred_element_type=jnp.float32) + c2[...])
    nfv = nf[...] + jnp.dot(h, v3[...], preferred_element_type=jnp.float32) + c3[...]
    nfo[...] = nfv
    a[...] = jnp.dot(nfv, wa[...], preferred_element_type=jnp.float32)
    b[...] = jnp.dot(nfv, wb[...], preferred_element_type=jnp.float32)


def _node_final_body(nf, p, v1a, v1b, c1, v2, c2, v3, c3,
                     o1, d1, o2, d2, o3, d3, out):
    aggr = p[0] + p[1]
    h = _lrelu(jnp.dot(nf[...], v1a[...], preferred_element_type=jnp.float32)
               + jnp.dot(aggr, v1b[...], preferred_element_type=jnp.float32) + c1[...])
    h = _lrelu(jnp.dot(h, v2[...], preferred_element_type=jnp.float32) + c2[...])
    nfv = nf[...] + jnp.dot(h, v3[...], preferred_element_type=jnp.float32) + c3[...]
    h = _lrelu(jnp.dot(nfv, o1[...], preferred_element_type=jnp.float32) + d1[...])
    h = _lrelu(jnp.dot(h, o2[...], preferred_element_type=jnp.float32) + d2[...])
    out[...] = jnp.dot(h, o3[...], preferred_element_type=jnp.float32) + d3[...]


def _const_spec(shape):
    return pl.BlockSpec(shape, lambda i: (0,) * len(shape))


def _edge_mlp_call(write_ef, g1, g2, ef, nd, w1c, b1, w2, b2, w3, b3):
    ne = g1.shape[0]
    grid = (ne // _BE,)
    blk = pl.BlockSpec((_BE, _H), lambda i: (i, 0))
    gblk = pl.BlockSpec((_BE, _H), lambda i: (i, 0))
    n_out = 2 if write_ef else 1
    res = pl.pallas_call(
        functools.partial(_edge_mlp_body, write_ef),
        grid=grid,
        in_specs=[
            gblk, gblk, blk,
            pl.BlockSpec((1, 1, _BE), lambda i: (i, 0, 0)),
            _const_spec(w1c.shape), _const_spec(b1.shape),
            _const_spec(w2.shape), _const_spec(b2.shape),
            _const_spec(w3.shape), _const_spec(b3.shape),
        ],
        out_specs=[blk] * n_out,
        out_shape=[jax.ShapeDtypeStruct((ne, _H), jnp.float32)] * n_out,
        compiler_params=pltpu.CompilerParams(
            dimension_semantics=("arbitrary",)),
    )(g1, g2, ef, nd, w1c, b1, w2, b2, w3, b3)
    return res if write_ef else res[0]


def _edge0_call(g1, g2, ea, nd, ei, lw):
    ne = g1.shape[0]
    grid = (ne // _BE,)
    blk = pl.BlockSpec((_BE, _H), lambda i: (i, 0))
    gblk = pl.BlockSpec((_BE, _H), lambda i: (i, 0))
    consts = [ei[0], ei[1], ei[2], ei[3], ei[4], ei[5],
              lw["w1c"], lw["b1"], lw["w2"], lw["b2"], lw["w3"], lw["b3"]]
    return pl.pallas_call(
        _edge0_body,
        grid=grid,
        in_specs=[
            gblk, gblk,
            pl.BlockSpec((_BE, ea.shape[1]), lambda i: (i, 0)),
            pl.BlockSpec((1, 1, _BE), lambda i: (i, 0, 0)),
        ] + [_const_spec(c.shape) for c in consts],
        out_specs=[blk, blk],
        out_shape=[jax.ShapeDtypeStruct((ne, _H), jnp.float32)] * 2,
        compiler_params=pltpu.CompilerParams(
            dimension_semantics=("arbitrary",)),
    )(g1, g2, ea, nd, *consts)


_UN = 5
_NSTR = 1
_ES = _E // _NSTR


def _sc_gather(a, b, dst, src):
    mesh = plsc.VectorSubcoreMesh(core_axis_name="c", subcore_axis_name="s")
    ne = dst.shape[0]
    epw = ne // _NW
    nch = epw // _CH
    nphase = nch // _UN

    gdt = a.dtype
    gw = a.shape[1]

    @functools.partial(
        pl.kernel,
        out_type=[jax.ShapeDtypeStruct((ne, gw), gdt),
                  jax.ShapeDtypeStruct((ne, gw), gdt)],
        mesh=mesh,
        scratch_types=(
            [pltpu.VMEM((epw,), jnp.int32)] * 2
            + [pltpu.VMEM((_CH, gw), gdt)] * (2 * _UN)
            + [pltpu.SemaphoreType.DMA] * (4 * _UN)
        ),
    )
    def gather_k(a_hbm, b_hbm, dst_hbm, src_hbm, g1_hbm, g2_hbm, *scr):
        idxd, idxs = scr[0], scr[1]
        bufa = scr[2:2 + _UN]
        bufb = scr[2 + _UN:2 + 2 * _UN]
        gsa = scr[2 + 2 * _UN:2 + 3 * _UN]
        gsb = scr[2 + 3 * _UN:2 + 4 * _UN]
        wsa = scr[2 + 4 * _UN:2 + 5 * _UN]
        wsb = scr[2 + 5 * _UN:2 + 6 * _UN]
        wid = lax.axis_index("s") * _NC + lax.axis_index("c")
        base = wid * epw
        pltpu.sync_copy(dst_hbm.at[pl.ds(base, epw)], idxd)
        pltpu.sync_copy(src_hbm.at[pl.ds(base, epw)], idxs)

        def fire(s, ch):
            pltpu.async_copy(a_hbm.at[idxd.at[pl.ds(ch * _CH, _CH)]],
                             bufa[s], gsa[s])
            pltpu.async_copy(b_hbm.at[idxs.at[pl.ds(ch * _CH, _CH)]],
                             bufb[s], gsb[s])

        def drain_and_write(s, ch):
            off = base + ch * _CH
            pltpu.make_async_copy(a_hbm.at[idxd.at[pl.ds(0, _CH)]],
                                  bufa[s], gsa[s]).wait()
            pltpu.make_async_copy(b_hbm.at[idxs.at[pl.ds(0, _CH)]],
                                  bufb[s], gsb[s]).wait()
            pltpu.async_copy(bufa[s], g1_hbm.at[pl.ds(off, _CH)], wsa[s])
            pltpu.async_copy(bufb[s], g2_hbm.at[pl.ds(off, _CH)], wsb[s])

        def drain_write(s):
            pltpu.make_async_copy(bufa[s], g1_hbm.at[pl.ds(base, _CH)],
                                  wsa[s]).wait()
            pltpu.make_async_copy(bufb[s], g2_hbm.at[pl.ds(base, _CH)],
                                  wsb[s]).wait()

        for s in range(_UN):
            fire(s, s)

        def body(j, carry):
            for s in range(_UN):
                drain_and_write(s, j * _UN + s)
            for s in range(_UN):
                drain_write(s)
                fire(s, (j + 1) * _UN + s)
            return carry

        lax.fori_loop(0, nphase - 1, body, 0)
        for s in range(_UN):
            drain_and_write(s, (nphase - 1) * _UN + s)
        for s in range(_UN):
            drain_write(s)

    return gather_k(a, b, dst, src)


def _sc_scatter(msgds, dsts, zeros):
    mesh = plsc.VectorSubcoreMesh(core_axis_name="c", subcore_axis_name="s")
    nstr = len(msgds)
    epc = _ES // _NC
    ept = epc // _NS
    nchunks = _N // _CH
    kmax = (nchunks + _NS - 1) // _NS

    ncht = ept // _CH
    UNS = 4
    nfull = ncht // UNS - 1

    @functools.partial(
        pl.kernel,
        out_type=jax.ShapeDtypeStruct((_NC, _N, _H), jnp.float32),
        mesh=mesh,
        scratch_types=(
            [pltpu.VMEM((_CH,), jnp.int32)] * UNS
            + [pltpu.VMEM((_CH, _H), jnp.float32)] * UNS
            + [pltpu.VMEM_SHARED((_N, _H), jnp.float32)]
            + [pltpu.SemaphoreType.DMA] * (3 * UNS + 3)
        ),
    )
    def scatter_k(*refs):
        msgd_hbm = refs[0:nstr]
        dst_hbm = refs[nstr:2 * nstr]
        zeros_hbm = refs[2 * nstr]
        part_hbm = refs[2 * nstr + 1]
        scr = refs[2 * nstr + 2:]
        idx = scr[0:UNS]
        buf = scr[UNS:2 * UNS]
        accum = scr[2 * UNS]
        isem = scr[2 * UNS + 1:3 * UNS + 1]
        dsem = scr[3 * UNS + 1:4 * UNS + 1]
        ssem = scr[4 * UNS + 1:5 * UNS + 1]
        zsem = scr[5 * UNS + 1]
        rsem = (scr[5 * UNS + 2], scr[5 * UNS + 3])
        cid = lax.axis_index("c")
        sid = lax.axis_index("s")

        pltpu.sync_copy(zeros_hbm, buf[0])
        for k in range(kmax):
            ch = sid + k * _NS

            @pl.when(ch < nchunks)
            def _():
                pltpu.async_copy(buf[0], accum.at[pl.ds(ch * _CH, _CH)], zsem)

        for k in range(kmax):
            ch = sid + k * _NS

            @pl.when(ch < nchunks)
            def _():
                pltpu.make_async_copy(buf[0], accum.at[pl.ds(0, _CH)],
                                      zsem).wait()

        plsc.subcore_barrier()

        ebase = cid * epc + sid * ept

        def run_stripe(mref, dref):
            def fire_load(s, ch):
                off = ebase + ch * _CH
                pltpu.async_copy(dref.at[pl.ds(off, _CH)], idx[s], isem[s])
                pltpu.async_copy(mref.at[pl.ds(off, _CH)], buf[s], dsem[s])

            def drain_and_scatter(s):
                pltpu.make_async_copy(dref.at[pl.ds(ebase, _CH)],
                                      idx[s], isem[s]).wait()
                pltpu.make_async_copy(mref.at[pl.ds(ebase, _CH)],
                                      buf[s], dsem[s]).wait()
                pltpu.async_copy(buf[s], accum.at[idx[s]], ssem[s], add=True)

            def drain_scatter(s):
                pltpu.make_async_copy(buf[s], accum.at[idx[s]], ssem[s]).wait()

            for s in range(UNS):
                fire_load(s, s)

            def body(j, carry):
                for s in range(UNS):
                    drain_and_scatter(s)
                for s in range(UNS):
                    drain_scatter(s)
                    fire_load(s, (j + 1) * UNS + s)
                return carry

            lax.fori_loop(0, nfull, body, 0)
            for s in range(UNS):
                drain_and_scatter(s)
            drain_scatter(0)
            fire_load(0, ncht - 1)
            drain_and_scatter(0)
            for s in range(UNS):
                drain_scatter(s)

        for st in range(nstr):
            run_stripe(msgd_hbm[st], dst_hbm[st])
        plsc.subcore_barrier()

        for k in range(kmax):
            ch = sid + k * _NS
            p = k % 2

            @pl.when(ch < nchunks)
            def _():
                r = ch * _CH
                if k >= 2:
                    pltpu.make_async_copy(
                        buf[p], part_hbm.at[cid].at[pl.ds(0, _CH)],
                        rsem[p]).wait()
                pltpu.sync_copy(accum.at[pl.ds(r, _CH)], buf[p])
                pltpu.async_copy(buf[p], part_hbm.at[cid].at[pl.ds(r, _CH)],
                                 rsem[p])

        for k in range(kmax):
            ch = sid + k * _NS

            @pl.when(jnp.logical_and(ch < nchunks,
                                     ch + 2 * _NS >= nchunks))
            def _():
                pltpu.make_async_copy(
                    buf[k % 2],
                    part_hbm.at[cid].at[pl.ds(0, _CH)], rsem[k % 2]).wait()

    return scatter_k(*msgds, *dsts, zeros)


def _rb(b):
    return b.reshape(1, -1)


def kernel(x, edge_index, edge_attr, node_dist, params):
    src = edge_index[0].astype(jnp.int32)
    dst = edge_index[1].astype(jnp.int32)
    x2 = x.astype(jnp.int32).reshape(_N, 1)
    emb = params["embed"]

    (niw1, nib1), (niw2, nib2), (niw3, nib3) = params["node_in"]
    (eiw1, eib1), (eiw2, eib2), (eiw3, eib3) = params["edge_in"]
    (now1, nod1), (now2, nod2), (now3, nod3) = params["node_out"]

    layer_w = []
    for lp in params["layers"]:
        (w1, b1), (w2, b2), (w3, b3) = lp["lin_edge"]
        (v1, c1), (v2, c2), (v3, c3) = lp["lin_node"]
        layer_w.append(dict(
            wa=w1[0:_H], wb=w1[_H:2 * _H], w1c=w1[2 * _H:3 * _H],
            b1=_rb(b1), w2=w2, b2=_rb(b2), w3=w3, b3=_rb(b3),
            v1a=v1[0:_H], v1b=v1[_H:2 * _H], c1=_rb(c1),
            v2=v2, c2=_rb(c2), v3=v3, c3=_rb(c3),
        ))

    _node_out3 = [jax.ShapeDtypeStruct((_N, _H), jnp.float32)] * 3
    nblk = pl.BlockSpec((_BN, _H), lambda i: (i, 0))
    ngrid = (_N // _BN,)
    ncp = pltpu.CompilerParams(dimension_semantics=("arbitrary",))

    ni_consts = [emb, niw1, _rb(nib1), niw2, _rb(nib2), niw3, _rb(nib3),
                 layer_w[0]["wa"], layer_w[0]["wb"]]
    nf, ga, gb = pl.pallas_call(
        _node_init_body,
        grid=ngrid,
        in_specs=[pl.BlockSpec((_BN, 1), lambda i: (i, 0))]
                 + [_const_spec(c.shape) for c in ni_consts],
        out_specs=[nblk] * 3,
        out_shape=_node_out3,
        compiler_params=ncp,
    )(x2, *ni_consts)

    zeros = jnp.zeros((_CH, _H), jnp.float32)
    nd = node_dist.astype(jnp.float32).reshape(_E // _BE, 1, _BE)

    dst_s = [lax.slice(dst, (i * _ES,), ((i + 1) * _ES,))
             for i in range(_NSTR)]
    src_s = [lax.slice(src, (i * _ES,), ((i + 1) * _ES,))
             for i in range(_NSTR)]
    ea_s = [lax.slice(edge_attr, (i * _ES, 0), ((i + 1) * _ES, edge_attr.shape[1]))
            for i in range(_NSTR)]
    nb = _ES // _BE
    nd_s = [lax.slice(nd, (i * nb, 0, 0), ((i + 1) * nb, 1, _BE))
            for i in range(_NSTR)]

    ei = (eiw1, _rb(eib1), eiw2, _rb(eib2), eiw3, _rb(eib3))
    ef_s = [None] * _NSTR

    for l, lw in enumerate(layer_w):
        last = l == len(layer_w) - 1
        g_s = [_sc_gather(ga, gb, dst_s[i], src_s[i]) for i in range(_NSTR)]
        msgd_s = []
        for i in range(_NSTR):
            g1, g2 = g_s[i]
            if l == 0:
                ef_s[i], msgd = _edge0_call(g1, g2, ea_s[i], nd_s[i], ei, lw)
            elif not last:
                ef_s[i], msgd = _edge_mlp_call(
                    True, g1, g2, ef_s[i], nd_s[i],
                    lw["w1c"], lw["b1"], lw["w2"], lw["b2"],
                    lw["w3"], lw["b3"])
            else:
                msgd = _edge_mlp_call(
                    False, g1, g2, ef_s[i], nd_s[i],
                    lw["w1c"], lw["b1"], lw["w2"], lw["b2"],
                    lw["w3"], lw["b3"])
            msgd_s.append(msgd)
        p = _sc_scatter(msgd_s, dst_s, zeros)
        pblk = pl.BlockSpec((_NC, _BN, _H), lambda i: (0, i, 0))
        if not last:
            nxt = layer_w[l + 1]
            nu_consts = [lw["v1a"], lw["v1b"], lw["c1"], lw["v2"], lw["c2"],
                         lw["v3"], lw["c3"], nxt["wa"], nxt["wb"]]
            nf, ga, gb = pl.pallas_call(
                _node_upd_body,
                grid=ngrid,
                in_specs=[nblk, pblk]
                         + [_const_spec(c.shape) for c in nu_consts],
                out_specs=[nblk] * 3,
                out_shape=_node_out3,
                compiler_params=ncp,
            )(nf, p, *nu_consts)
        else:
            nfin_consts = [lw["v1a"], lw["v1b"], lw["c1"], lw["v2"], lw["c2"],
                           lw["v3"], lw["c3"], now1, _rb(nod1), now2,
                           _rb(nod2), now3, _rb(nod3)]
            out = pl.pallas_call(
                _node_final_body,
                grid=ngrid,
                in_specs=[nblk, pblk]
                         + [_const_spec(c.shape) for c in nfin_consts],
                out_specs=pl.BlockSpec((_BN, now3.shape[1]), lambda i: (i, 0)),
                out_shape=jax.ShapeDtypeStruct((_N, now3.shape[1]), jnp.float32),
                compiler_params=ncp,
            )(nf, p, *nfin_consts)
    return out
